# baseline (speedup 1.0000x reference)
import functools

import jax
import jax.numpy as jnp
import numpy as np
from jax import lax
from jax.experimental import pallas as pl
from jax.experimental.pallas import tpu as pltpu
from jax.experimental.pallas import tpu_sc as plsc

_TABK = 128
_C1 = 1.4426950408889634
_C2 = -0.7213475204444817
_SOFTPLUS = (0.693120002746582, -0.4994899034500122, 0.1226591020822525,
             0.004528840072453022, -0.009686391800642014,
             0.0023859606590121984, -0.00026507495203986764,
             1.1840770639537368e-05, -1.7712237010414356e-08)


def _table_const():
    ar = np.arange(_TABK)
    return np.concatenate([
        np.log2(1.0 + ar / _TABK),
        1.0 / (1.0 + ar / _TABK),
    ]).astype(np.float32)


def _log2_tab(x, tab_buf):
    i = lax.bitcast_convert_type(x, jnp.int32)
    e = (i >> 23) - 127
    k = (i >> 16) & 0x7F
    m = lax.bitcast_convert_type(
        (i & 0x007FFFFF) | 0x3F800000, jnp.float32)
    t = plsc.load_gather(tab_buf, [k])
    inv = plsc.load_gather(tab_buf, [k + _TABK])
    r = m * inv - jnp.float32(1.0)
    return (e.astype(jnp.float32) + t) + r * (jnp.float32(_C1)
                                              + jnp.float32(_C2) * r)


def _pair_compute(l0, l1, u0, u1, tab_buf):
    d = l1 - l0
    td = jnp.exp(d)
    act = _log2_tab(u1, tab_buf) > _log2_tab(u0, tab_buf) * td
    xx = jnp.minimum(jnp.abs(d), jnp.float32(5.0))
    lp1 = jnp.float32(_SOFTPLUS[-1])
    for c in _SOFTPLUS[-2::-1]:
        lp1 = lp1 * xx + jnp.float32(c)
    logp = jnp.minimum(jnp.where(act, d, -d), jnp.float32(0.0)) - lp1
    return jnp.where(act, jnp.int32(1), jnp.int32(0)), logp


def _make_sc_kernel(n_b: int, n_s: int):
    info = plsc.get_sparse_core_info()
    nc, ns, lanes = info.num_cores, info.num_subcores, info.num_lanes
    nw = nc * ns
    n_bblk = n_b // 8
    n_sblk = n_s // 128
    sh_per_b = nw // n_bblk
    sblk_per_w = n_sblk // sh_per_b
    kblk = 8
    n_chunks = sblk_per_w // kblk
    in_run = kblk * 256
    in_words = 8 * in_run
    out_words = kblk * 1024
    n_words = n_b * n_s * 2

    mesh = plsc.VectorSubcoreMesh(core_axis_name="c", subcore_axis_name="s")

    @functools.partial(
        pl.kernel,
        mesh=mesh,
        compiler_params=pltpu.CompilerParams(needs_layout_passes=False),
        out_type=[
            jax.ShapeDtypeStruct((n_b * n_s,), jnp.int32),
            jax.ShapeDtypeStruct((n_b * n_s,), jnp.float32),
        ],
        scratch_types=[
            pltpu.VMEM((in_words,), jnp.float32),
            pltpu.VMEM((in_words,), jnp.float32),
            pltpu.VMEM((in_words,), jnp.float32),
            pltpu.VMEM((in_words,), jnp.float32),
            pltpu.VMEM((out_words,), jnp.int32),
            pltpu.VMEM((out_words,), jnp.int32),
            pltpu.VMEM((out_words,), jnp.float32),
            pltpu.VMEM((out_words,), jnp.float32),
            pltpu.VMEM((2 * _TABK,), jnp.float32),
            pltpu.SemaphoreType.DMA,
            pltpu.SemaphoreType.DMA,
            pltpu.SemaphoreType.DMA,
            pltpu.SemaphoreType.DMA,
        ],
    )
    def k(l_hbm, u_hbm, tab_hbm, act_hbm, lp_hbm,
          l_b0, l_b1, u_b0, u_b1, a_b0, a_b1, p_b0, p_b1, tab_buf,
          sem_in0, sem_in1, sem_out0, sem_out1):
        pltpu.sync_copy(tab_hbm, tab_buf)
        wid = lax.axis_index("s") * nc + lax.axis_index("c")
        bb = wid // sh_per_b
        sh = wid % sh_per_b
        sblk_base = sh * sblk_per_w

        l_bufs, u_bufs = (l_b0, l_b1), (u_b0, u_b1)
        a_bufs, p_bufs = (a_b0, a_b1), (p_b0, p_b1)
        sem_in, sem_out = (sem_in0, sem_in1), (sem_out0, sem_out1)

        def in_copies(c, slot):
            sblk0 = sblk_base + c * kblk
            for bi in range(8):
                src = ((bb * 8 + bi) * n_sblk + sblk0) * 256
                dst = pl.ds(bi * in_run, in_run)
                yield pltpu.make_async_copy(
                    l_hbm.at[pl.ds(src, in_run)], l_bufs[slot].at[dst],
                    sem_in[slot])
                yield pltpu.make_async_copy(
                    u_hbm.at[pl.ds(src, in_run)], u_bufs[slot].at[dst],
                    sem_in[slot])

        def out_copies(c, slot):
            dst = pl.ds((bb * n_sblk + sblk_base + c * kblk) * 1024, out_words)
            yield pltpu.make_async_copy(a_bufs[slot], act_hbm.at[dst],
                                        sem_out[slot])
            yield pltpu.make_async_copy(p_bufs[slot], lp_hbm.at[dst],
                                        sem_out[slot])

        def issue(copies):
            for cp in copies:
                cp.start()

        def drain(copies):
            for cp in copies:
                cp.wait()

        def make_inner(slot):
            l_buf, u_buf = l_bufs[slot], u_bufs[slot]
            act_buf, lp_buf = a_bufs[slot], p_bufs[slot]

            def inner(t):
                bi = t >> 6
                sb = (t >> 3) & 7
                j = t & 7
                ioff = bi * in_run + sb * 256 + j * lanes
                ooff = sb * 1024 + bi * 128 + j * lanes
                l0 = l_buf[pl.ds(ioff, lanes)]
                l1 = l_buf[pl.ds(ioff + 128, lanes)]
                u0 = u_buf[pl.ds(ioff, lanes)]
                u1 = u_buf[pl.ds(ioff + 128, lanes)]
                act, logp = _pair_compute(l0, l1, u0, u1, tab_buf)
                act_buf[pl.ds(ooff, lanes)] = act
                lp_buf[pl.ds(ooff, lanes)] = logp

            return inner

        n_iters = 8 * kblk * 8

        issue(in_copies(0, 0))

        def pair_body(kk, carry):
            c0 = 2 * kk
            c1 = c0 + 1
            issue(in_copies(c1, 1))
            drain(in_copies(c0, 0))

            @pl.when(kk > 0)
            def _():
                drain(out_copies(c0 - 2, 0))

            plsc.parallel_loop(0, n_iters, unroll=4)(make_inner(0))
            issue(out_copies(c0, 0))

            @pl.when(kk < n_chunks // 2 - 1)
            def _():
                issue(in_copies(c0 + 2, 0))

            drain(in_copies(c1, 1))

            @pl.when(kk > 0)
            def _():
                drain(out_copies(c1 - 2, 1))

            plsc.parallel_loop(0, n_iters, unroll=4)(make_inner(1))
            issue(out_copies(c1, 1))
            return carry

        lax.fori_loop(0, n_chunks // 2, pair_body, 0)
        drain(out_copies(n_chunks - 2, 0))
        drain(out_copies(n_chunks - 1, 1))

    return k


def kernel(logits, gumbel_u):
    b, s, a = logits.shape
    assert a == 2 and b % 8 == 0 and s % 128 == 0
    sc = _make_sc_kernel(b, s)

    def to_view(x):
        return x.reshape(b, s // 128, 128, 2).transpose(0, 1, 3, 2).reshape(-1)

    acts_f, lp_f = sc(to_view(logits), to_view(gumbel_u),
                      jnp.asarray(_table_const()))

    def from_view(x):
        return (x.reshape(b // 8, s // 128, 8, 128)
                .transpose(0, 2, 1, 3).reshape(b, s))

    return from_view(acts_f), from_view(lp_f)


# speedup vs baseline: 1.0729x; 1.0729x over previous
import functools

import jax
import jax.numpy as jnp
import numpy as np
from jax import lax
from jax.experimental import pallas as pl
from jax.experimental.pallas import tpu as pltpu
from jax.experimental.pallas import tpu_sc as plsc

_TABK = 128
_C1 = 1.4426950408889634
_C2 = -0.7213475204444817
_LOG1P = (2.211703031207435e-05, 0.999010443687439, -0.4891568422317505,
          0.2833043336868286, -0.1301194131374359, 0.030102625489234924)


def _table_const():
    ar = np.arange(_TABK)
    return np.concatenate([
        np.log2(1.0 + ar / _TABK),
        1.0 / (1.0 + ar / _TABK),
    ]).astype(np.float32)


def _log2_tab(x, tab_buf):
    i = lax.bitcast_convert_type(x, jnp.int32)
    e = (i >> 23) - 127
    k = (i >> 16) & 0x7F
    m = lax.bitcast_convert_type(
        (i & 0x007FFFFF) | 0x3F800000, jnp.float32)
    t = plsc.load_gather(tab_buf, [k])
    inv = plsc.load_gather(tab_buf, [k + _TABK])
    r = m * inv - jnp.float32(1.0)
    return (e.astype(jnp.float32) + t) + r * (jnp.float32(_C1)
                                              + jnp.float32(_C2) * r)


def _pair_compute(l0, l1, u0, u1, tab_buf):
    d = l1 - l0
    td = jnp.exp(d)
    act = _log2_tab(u1, tab_buf) > _log2_tab(u0, tab_buf) * td
    emd = jnp.exp(-jnp.abs(d))
    lp1 = jnp.float32(_LOG1P[-1])
    for c in _LOG1P[-2::-1]:
        lp1 = lp1 * emd + jnp.float32(c)
    logp = jnp.minimum(jnp.where(act, d, -d), jnp.float32(0.0)) - lp1
    return jnp.where(act, jnp.int32(1), jnp.int32(0)), logp


def _make_sc_kernel(n_b: int, n_s: int):
    info = plsc.get_sparse_core_info()
    nc, ns, lanes = info.num_cores, info.num_subcores, info.num_lanes
    nw = nc * ns
    n_bblk = n_b // 8
    n_sblk = n_s // 128
    sh_per_b = nw // n_bblk
    sblk_per_w = n_sblk // sh_per_b
    kblk = 4
    n_chunks = sblk_per_w // kblk
    in_run = kblk * 256
    in_words = 8 * in_run
    out_words = kblk * 1024
    n_words = n_b * n_s * 2

    mesh = plsc.VectorSubcoreMesh(core_axis_name="c", subcore_axis_name="s")

    @functools.partial(
        pl.kernel,
        mesh=mesh,
        compiler_params=pltpu.CompilerParams(needs_layout_passes=False),
        out_type=[
            jax.ShapeDtypeStruct((n_b * n_s,), jnp.int32),
            jax.ShapeDtypeStruct((n_b * n_s,), jnp.float32),
        ],
        scratch_types=[
            pltpu.VMEM((in_words,), jnp.float32),
            pltpu.VMEM((in_words,), jnp.float32),
            pltpu.VMEM((in_words,), jnp.float32),
            pltpu.VMEM((in_words,), jnp.float32),
            pltpu.VMEM((out_words,), jnp.int32),
            pltpu.VMEM((out_words,), jnp.int32),
            pltpu.VMEM((out_words,), jnp.float32),
            pltpu.VMEM((out_words,), jnp.float32),
            pltpu.VMEM((2 * _TABK,), jnp.float32),
            pltpu.SemaphoreType.DMA,
            pltpu.SemaphoreType.DMA,
            pltpu.SemaphoreType.DMA,
            pltpu.SemaphoreType.DMA,
        ],
    )
    def k(l_hbm, u_hbm, tab_hbm, act_hbm, lp_hbm,
          l_b0, l_b1, u_b0, u_b1, a_b0, a_b1, p_b0, p_b1, tab_buf,
          sem_in0, sem_in1, sem_out0, sem_out1):
        pltpu.sync_copy(tab_hbm, tab_buf)
        wid = lax.axis_index("s") * nc + lax.axis_index("c")
        bb = wid // sh_per_b
        sh = wid % sh_per_b
        sblk_base = sh * sblk_per_w

        l_bufs, u_bufs = (l_b0, l_b1), (u_b0, u_b1)
        a_bufs, p_bufs = (a_b0, a_b1), (p_b0, p_b1)
        sem_in, sem_out = (sem_in0, sem_in1), (sem_out0, sem_out1)

        def in_copies(c, slot):
            sblk0 = sblk_base + c * kblk
            for bi in range(8):
                src = ((bb * 8 + bi) * n_sblk + sblk0) * 256
                dst = pl.ds(bi * in_run, in_run)
                yield pltpu.make_async_copy(
                    l_hbm.at[pl.ds(src, in_run)], l_bufs[slot].at[dst],
                    sem_in[slot])
                yield pltpu.make_async_copy(
                    u_hbm.at[pl.ds(src, in_run)], u_bufs[slot].at[dst],
                    sem_in[slot])

        def out_copies(c, slot):
            dst = pl.ds((bb * n_sblk + sblk_base + c * kblk) * 1024, out_words)
            yield pltpu.make_async_copy(a_bufs[slot], act_hbm.at[dst],
                                        sem_out[slot])
            yield pltpu.make_async_copy(p_bufs[slot], lp_hbm.at[dst],
                                        sem_out[slot])

        def issue(copies):
            for cp in copies:
                cp.start()

        def drain(copies):
            for cp in copies:
                cp.wait()

        def make_inner(slot):
            l_buf, u_buf = l_bufs[slot], u_bufs[slot]
            act_buf, lp_buf = a_bufs[slot], p_bufs[slot]

            def inner(t):
                bi = t >> 6
                sb = (t >> 3) & 7
                j = t & 7
                ioff = bi * in_run + sb * 256 + j * lanes
                ooff = sb * 1024 + bi * 128 + j * lanes
                l0 = l_buf[pl.ds(ioff, lanes)]
                l1 = l_buf[pl.ds(ioff + 128, lanes)]
                u0 = u_buf[pl.ds(ioff, lanes)]
                u1 = u_buf[pl.ds(ioff + 128, lanes)]
                act, logp = _pair_compute(l0, l1, u0, u1, tab_buf)
                act_buf[pl.ds(ooff, lanes)] = act
                lp_buf[pl.ds(ooff, lanes)] = logp

            return inner

        n_iters = 8 * kblk * 8

        issue(in_copies(0, 0))

        def pair_body(kk, carry):
            c0 = 2 * kk
            c1 = c0 + 1
            issue(in_copies(c1, 1))
            drain(in_copies(c0, 0))

            @pl.when(kk > 0)
            def _():
                drain(out_copies(c0 - 2, 0))

            plsc.parallel_loop(0, n_iters, unroll=4)(make_inner(0))
            issue(out_copies(c0, 0))

            @pl.when(kk < n_chunks // 2 - 1)
            def _():
                issue(in_copies(c0 + 2, 0))

            drain(in_copies(c1, 1))

            @pl.when(kk > 0)
            def _():
                drain(out_copies(c1 - 2, 1))

            plsc.parallel_loop(0, n_iters, unroll=4)(make_inner(1))
            issue(out_copies(c1, 1))
            return carry

        lax.fori_loop(0, n_chunks // 2, pair_body, 0)
        drain(out_copies(n_chunks - 2, 0))
        drain(out_copies(n_chunks - 1, 1))

    return k


def kernel(logits, gumbel_u):
    b, s, a = logits.shape
    assert a == 2 and b % 8 == 0 and s % 128 == 0
    sc = _make_sc_kernel(b, s)

    def to_view(x):
        return x.reshape(b, s // 128, 128, 2).transpose(0, 1, 3, 2).reshape(-1)

    acts_f, lp_f = sc(to_view(logits), to_view(gumbel_u),
                      jnp.asarray(_table_const()))

    def from_view(x):
        return (x.reshape(b // 8, s // 128, 8, 128)
                .transpose(0, 2, 1, 3).reshape(b, s))

    return from_view(acts_f), from_view(lp_f)
